# Initial kernel scaffold; baseline (speedup 1.0000x reference)
import jax
import jax.numpy as jnp
from jax.experimental import pallas as pl


def kernel(x, w1, gamma1, beta1, w2, gamma2, beta2):
    raise NotImplementedError("write your pallas kernel here")


# speedup vs baseline: 1.4699x; 1.4699x over previous
import jax
import jax.numpy as jnp
from jax.experimental import pallas as pl
from jax.experimental.pallas import tpu as pltpu


def _basic_block(x, w1, gamma1, beta1, w2, gamma2, beta2, eps=1e-5):
    N, C, H, W = x.shape
    Cout = w1.shape[0]
    f32 = jnp.float32
    bf16 = jnp.bfloat16
    Wp = W + 2
    Hf = H * Wp
    Lp = (H + 3) * Wp
    K9 = 9 * C
    offs = [kh * Wp + kw for kh in range(3) for kw in range(3)]

    xpf = jnp.pad(x, ((0, 0), (0, 0), (1, 2), (1, 1))).reshape(N, C, Lp)
    w1s = jnp.transpose(w1, (0, 2, 3, 1)).reshape(Cout, K9).astype(bf16)
    w2s = jnp.transpose(w2, (0, 2, 3, 1)).reshape(Cout, K9).astype(bf16)
    mask = (jnp.arange(Hf, dtype=jnp.int32) % Wp < W).astype(f32).reshape(1, Hf)

    def p1_kernel(xpf_ref, w1s_ref, mask_ref, y1_ref, ssum_ref, ssq_ref,
                  xbf_ref, xs_ref):
        xbf_ref[...] = xpf_ref[0].astype(bf16)
        for k, off in enumerate(offs):
            xs_ref[k * C:(k + 1) * C, :] = xbf_ref[:, off:off + Hf]
        y1 = jnp.dot(w1s_ref[...], xs_ref[...], preferred_element_type=f32)
        m = mask_ref[...]
        y1m = y1 * m
        ssum_ref[0] = jnp.sum(y1m, axis=1, keepdims=True)
        ssq_ref[0] = jnp.sum(y1m * y1, axis=1, keepdims=True)
        y1_ref[0] = y1.astype(bf16)

    def p2_kernel(y1b_ref, w2s_ref, mask_ref, s1_ref, b1_ref,
                  y2_ref, ssum_ref, ssq_ref, apad_ref, as_ref):
        m = mask_ref[...]
        a1 = jnp.maximum(y1b_ref[0].astype(f32) * s1_ref[...] + b1_ref[...],
                         0.0) * m
        apad_ref[:, :W + 3] = jnp.zeros((Cout, W + 3), bf16)
        apad_ref[:, W + 3 + Hf:] = jnp.zeros((Cout, Lp - W - 3 - Hf), bf16)
        apad_ref[:, W + 3:W + 3 + Hf] = a1.astype(bf16)
        for k, off in enumerate(offs):
            as_ref[k * C:(k + 1) * C, :] = apad_ref[:, off:off + Hf]
        y2 = jnp.dot(w2s_ref[...], as_ref[...], preferred_element_type=f32)
        y2m = y2 * m
        ssum_ref[0] = jnp.sum(y2m, axis=1, keepdims=True)
        ssq_ref[0] = jnp.sum(y2m * y2, axis=1, keepdims=True)
        y2_ref[0] = y2.astype(bf16)

    def p3_kernel(y2b_ref, xpf_ref, s2_ref, b2_ref, o_ref):
        xres = xpf_ref[0][:, W + 3:W + 3 + Hf]
        o_ref[0] = jnp.maximum(
            y2b_ref[0].astype(f32) * s2_ref[...] + b2_ref[...] + xres, 0.0)

    xpf_spec = pl.BlockSpec((1, C, Lp), lambda b: (b, 0, 0))
    ws_spec = pl.BlockSpec((Cout, K9), lambda b: (0, 0))
    mask_spec = pl.BlockSpec((1, Hf), lambda b: (0, 0))
    vec_spec = pl.BlockSpec((Cout, 1), lambda b: (0, 0))
    part_spec = pl.BlockSpec((1, Cout, 1), lambda b: (b, 0, 0))
    act_spec = pl.BlockSpec((1, Cout, Hf), lambda b: (b, 0, 0))
    part_shape = jax.ShapeDtypeStruct((N, Cout, 1), f32)
    act_shape = jax.ShapeDtypeStruct((N, Cout, Hf), bf16)
    parallel = pltpu.CompilerParams(dimension_semantics=("parallel",))

    cnt = jnp.asarray(N * H * W, f32)

    def _fold_bn(ssum, ssq, gamma, beta):
        mean = jnp.sum(ssum, axis=0) / cnt
        var = jnp.maximum(jnp.sum(ssq, axis=0) / cnt - mean * mean, 0.0)
        scale = gamma.reshape(-1, 1).astype(f32) / jnp.sqrt(var + eps)
        shift = beta.reshape(-1, 1).astype(f32) - mean * scale
        return scale, shift

    y1b, ssum1, ssq1 = pl.pallas_call(
        p1_kernel,
        out_shape=(act_shape, part_shape, part_shape),
        grid=(N,),
        in_specs=[xpf_spec, ws_spec, mask_spec],
        out_specs=(act_spec, part_spec, part_spec),
        scratch_shapes=[pltpu.VMEM((C, Lp), bf16), pltpu.VMEM((K9, Hf), bf16)],
        compiler_params=parallel,
    )(xpf, w1s, mask)
    s1, sh1 = _fold_bn(ssum1, ssq1, gamma1, beta1)

    y2b, ssum2, ssq2 = pl.pallas_call(
        p2_kernel,
        out_shape=(act_shape, part_shape, part_shape),
        grid=(N,),
        in_specs=[act_spec, ws_spec, mask_spec, vec_spec, vec_spec],
        out_specs=(act_spec, part_spec, part_spec),
        scratch_shapes=[pltpu.VMEM((Cout, Lp), bf16),
                        pltpu.VMEM((K9, Hf), bf16)],
        compiler_params=parallel,
    )(y1b, w2s, mask, s1, sh1)
    s2, sh2 = _fold_bn(ssum2, ssq2, gamma2, beta2)

    out_flat = pl.pallas_call(
        p3_kernel,
        out_shape=jax.ShapeDtypeStruct((N, Cout, Hf), x.dtype),
        grid=(N,),
        in_specs=[act_spec, xpf_spec, vec_spec, vec_spec],
        out_specs=pl.BlockSpec((1, Cout, Hf), lambda b: (b, 0, 0)),
        compiler_params=parallel,
    )(y2b, xpf, s2, sh2)

    return out_flat.reshape(N, Cout, H, Wp)[:, :, :, :W]


def kernel(x, w1, gamma1, beta1, w2, gamma2, beta2):
    return _basic_block(x, w1, gamma1, beta1, w2, gamma2, beta2)
